# Initial kernel scaffold; baseline (speedup 1.0000x reference)
import jax
import jax.numpy as jnp
from jax.experimental import pallas as pl


def kernel(x, edge_index, W, b):
    raise NotImplementedError("write your pallas kernel here")


# speedup vs baseline: 7.6392x; 7.6392x over previous
import functools

import jax
import jax.numpy as jnp
from jax import lax
from jax.experimental import pallas as pl
from jax.experimental.pallas import tpu as pltpu
from jax.experimental.pallas import tpu_sc as plsc

N = 10000
E = 320000
F = 128
NC = 2
NS = 16
L = 16

CW = 128
KJ = 10
GE = CW * KJ
EROWS = E // CW
DEG_GROUPS = E // GE
MSG_GROUPS = (E // NC) // GE
NCH = 80
NODE_CHUNKS = N // NCH


def _iota16():
    return lax.iota(jnp.int32, 16)


def _rsqrt16(d):
    i = plsc.bitcast(d, jnp.int32)
    i = jnp.int32(0x5F3759DF) - lax.shift_right_logical(i, 1)
    y = plsc.bitcast(i, jnp.float32)
    for _ in range(3):
        y = y * (1.5 - 0.5 * d * y * y)
    return y


def _sc_body(x_hbm, src_hbm, dst_hbm, part_hbm, feat_hbm,
             agg_sh, deg_sh,
             idxs_v, idxd_v, rows_v, xbuf_v, ones_s_v, ones_d_v, zdeg_v,
             dva_v, nsrc_v, ndst_v, sem):
    cid = lax.axis_index("c")
    sid = lax.axis_index("s")
    zf16 = jnp.zeros((16,), jnp.float32)
    zi16 = jnp.zeros((16,), jnp.int32)

    def _zero_row(r, _):
        for j in range(F // L):
            xbuf_v[r, pl.ds(j * L, L)] = zf16
        zdeg_v[r, :] = zf16
        return 0
    lax.fori_loop(0, NCH, _zero_row, 0)

    onesrow_s = jnp.where(_iota16() == 0, 1.0, 0.0).astype(jnp.float32)
    onesrow_d = jnp.where(_iota16() == 1, 1.0, 0.0).astype(jnp.float32)

    def _ones_row(r, _):
        ones_s_v[r, :] = onesrow_s
        ones_d_v[r, :] = onesrow_d
        return 0
    lax.fori_loop(0, CW, _ones_row, 0)

    trip_n = (NODE_CHUNKS - sid + NS - 1) // NS

    def _zero_spmem(k, _):
        base = (sid + NS * k) * NCH
        pltpu.sync_copy(xbuf_v, agg_sh.at[pl.ds(base, NCH)])
        pltpu.sync_copy(zdeg_v, deg_sh.at[pl.ds(base, NCH)])
        return 0
    lax.fori_loop(0, trip_n, _zero_spmem, 0)
    plsc.subcore_barrier()

    trip_d = (DEG_GROUPS - sid + NS - 1) // NS

    def _deg_group(k, _):
        row0 = (sid + NS * k) * KJ
        pltpu.sync_copy(src_hbm.at[pl.ds(row0, KJ)], idxs_v)
        pltpu.sync_copy(dst_hbm.at[pl.ds(row0, KJ)], idxd_v)
        for j in range(KJ):
            pltpu.sync_copy(ones_s_v, deg_sh.at[idxs_v.at[j]], add=True)
            pltpu.sync_copy(ones_d_v, deg_sh.at[idxd_v.at[j]], add=True)
        return 0
    lax.fori_loop(0, trip_d, _deg_group, 0)
    plsc.subcore_barrier()

    def _node_chunk(k, _):
        base = (sid + NS * k) * NCH
        pltpu.sync_copy(deg_sh.at[pl.ds(base, NCH)], dva_v)
        for v in range(NCH // L):
            ridx = _iota16() + v * L
            do = plsc.load_gather(dva_v, [ridx, zi16])
            nsrc_v[pl.ds(k * NCH + v * L, L)] = _rsqrt16(jnp.maximum(do, 1.0))
            di = plsc.load_gather(dva_v, [ridx, zi16 + 1])
            ndst_v[pl.ds(k * NCH + v * L, L)] = _rsqrt16(jnp.maximum(di, 1.0))
        pltpu.sync_copy(x_hbm.at[pl.ds(base, NCH)], xbuf_v)

        def _scale_grp(g, _):
            nv = nsrc_v[pl.ds(k * NCH + g * L, L)]
            for lane in range(L):
                s = nv[lane]
                r = g * L + lane
                for j in range(F // L):
                    xbuf_v[r, pl.ds(j * L, L)] = xbuf_v[r, pl.ds(j * L, L)] * s
            return 0
        lax.fori_loop(0, NCH // L, _scale_grp, 0)
        pltpu.sync_copy(xbuf_v, feat_hbm.at[pl.ds(base, NCH)])
        return 0
    lax.fori_loop(0, trip_n, _node_chunk, 0)
    plsc.subcore_barrier()

    trip_m = (MSG_GROUPS - sid + NS - 1) // NS
    erow0 = cid * (E // NC // CW)

    def _msg_group(k, _):
        row0 = erow0 + (sid + NS * k) * KJ
        pltpu.sync_copy(src_hbm.at[pl.ds(row0, KJ)], idxs_v)
        pltpu.sync_copy(dst_hbm.at[pl.ds(row0, KJ)], idxd_v)
        for j in range(KJ):
            pltpu.async_copy(feat_hbm.at[idxs_v.at[j]], rows_v, sem).wait()
            pltpu.sync_copy(rows_v, agg_sh.at[idxd_v.at[j]], add=True)
        return 0
    lax.fori_loop(0, trip_m, _msg_group, 0)
    plsc.subcore_barrier()

    def _out_chunk(k, _):
        base = (sid + NS * k) * NCH
        pltpu.sync_copy(agg_sh.at[pl.ds(base, NCH)], xbuf_v)

        def _scale_grp(g, _):
            nv = ndst_v[pl.ds(k * NCH + g * L, L)]
            for lane in range(L):
                s = nv[lane]
                r = g * L + lane
                for j in range(F // L):
                    xbuf_v[r, pl.ds(j * L, L)] = xbuf_v[r, pl.ds(j * L, L)] * s
            return 0
        lax.fori_loop(0, NCH // L, _scale_grp, 0)
        pltpu.sync_copy(xbuf_v, part_hbm.at[pl.ds(cid * N + base, NCH)])
        return 0
    lax.fori_loop(0, trip_n, _out_chunk, 0)


_sc_call = pl.kernel(
    _sc_body,
    out_type=(
        jax.ShapeDtypeStruct((NC * N, F), jnp.float32),
        jax.ShapeDtypeStruct((N, F), jnp.float32),
    ),
    mesh=plsc.VectorSubcoreMesh(core_axis_name="c", subcore_axis_name="s"),
    compiler_params=pltpu.CompilerParams(
        use_tc_tiling_on_sc=False, needs_layout_passes=False),
    scratch_types=[
        pltpu.VMEM_SHARED((N, F), jnp.float32),
        pltpu.VMEM_SHARED((N, 16), jnp.float32),
        pltpu.VMEM((KJ, CW), jnp.int32),
        pltpu.VMEM((KJ, CW), jnp.int32),
        pltpu.VMEM((CW, F), jnp.float32),
        pltpu.VMEM((NCH, F), jnp.float32),
        pltpu.VMEM((CW, 16), jnp.float32),
        pltpu.VMEM((CW, 16), jnp.float32),
        pltpu.VMEM((NCH, 16), jnp.float32),
        pltpu.VMEM((NCH, 16), jnp.float32),
        pltpu.VMEM((NODE_CHUNKS // NS * NCH + NCH,), jnp.float32),
        pltpu.VMEM((NODE_CHUNKS // NS * NCH + NCH,), jnp.float32),
        pltpu.SemaphoreType.DMA,
    ],
)


def _tc_body(p0_ref, p1_ref, w_ref, b_ref, o_ref):
    acc = p0_ref[...] + p1_ref[...]
    y = jnp.dot(acc, w_ref[...], preferred_element_type=jnp.float32,
                precision=lax.Precision.HIGHEST)
    o_ref[...] = jax.nn.sigmoid(y + b_ref[...])


@jax.jit
def kernel(x, edge_index, W, b):
    src2d = edge_index[0].reshape(EROWS, CW)
    dst2d = edge_index[1].reshape(EROWS, CW)
    part, _ = _sc_call(x, src2d, dst2d)
    out = pl.pallas_call(
        _tc_body,
        grid=(10,),
        in_specs=[
            pl.BlockSpec((N // 10, F), lambda i: (i, 0)),
            pl.BlockSpec((N // 10, F), lambda i: (i, 0)),
            pl.BlockSpec((F, F), lambda i: (0, 0)),
            pl.BlockSpec((1, F), lambda i: (0, 0)),
        ],
        out_specs=pl.BlockSpec((N // 10, F), lambda i: (i, 0)),
        out_shape=jax.ShapeDtypeStruct((N, F), jnp.float32),
    )(part[:N], part[N:], W, b.reshape(1, F))
    return out


# speedup vs baseline: 9.9446x; 1.3018x over previous
import functools

import jax
import jax.numpy as jnp
from jax import lax
from jax.experimental import pallas as pl
from jax.experimental.pallas import tpu as pltpu
from jax.experimental.pallas import tpu_sc as plsc

N = 10000
E = 320000
F = 128
NC = 2
NS = 16
L = 16

CW = 128
KJ = 10
GE = CW * KJ
EROWS = E // CW
DEG_GROUPS = E // GE
MSG_GROUPS = (E // NC) // GE
NCH = 80
NODE_CHUNKS = N // NCH
DW = 8


def _iota16():
    return lax.iota(jnp.int32, 16)


def _rsqrt16(d):
    i = plsc.bitcast(d, jnp.int32)
    i = jnp.int32(0x5F3759DF) - lax.shift_right_logical(i, 1)
    y = plsc.bitcast(i, jnp.float32)
    for _ in range(3):
        y = y * (1.5 - 0.5 * d * y * y)
    return y


def _sc_body(x_hbm, src_hbm, dst_hbm, part_hbm, feat_hbm,
             agg_sh, deg_sh,
             idxs_v, idxd_v, rows_a_v, rows_b_v, ones_s_v, ones_d_v,
             zdeg_v, nsrc_v, ndst_v, sem_g, sem_s):
    cid = lax.axis_index("c")
    sid = lax.axis_index("s")
    zf16 = jnp.zeros((16,), jnp.float32)
    zi16 = jnp.zeros((16,), jnp.int32)

    def _zero_row(r, _):
        for j in range(F // L):
            rows_a_v[r, pl.ds(j * L, L)] = zf16
        return 0
    lax.fori_loop(0, NCH, _zero_row, 0)

    def _init_narrow(g, _):
        p = _iota16() + g * L
        r = lax.shift_right_logical(p, 3)
        c = jnp.bitwise_and(p, 7)
        one_s = jnp.where(c == 0, 1.0, 0.0).astype(jnp.float32)
        one_d = jnp.where(c == 1, 1.0, 0.0).astype(jnp.float32)
        plsc.store_scatter(ones_s_v, [r, c], one_s)
        plsc.store_scatter(ones_d_v, [r, c], one_d)
        return 0
    lax.fori_loop(0, CW * DW // L, _init_narrow, 0)

    def _zero_narrow(g, _):
        p = _iota16() + g * L
        r = lax.shift_right_logical(p, 3)
        c = jnp.bitwise_and(p, 7)
        plsc.store_scatter(zdeg_v, [r, c], zf16)
        return 0
    lax.fori_loop(0, NCH * DW // L, _zero_narrow, 0)

    trip_n = (NODE_CHUNKS - sid + NS - 1) // NS

    def _zero_spmem(k, _):
        base = (sid + NS * k) * NCH
        pltpu.sync_copy(rows_a_v.at[pl.ds(0, NCH)], agg_sh.at[pl.ds(base, NCH)])
        pltpu.sync_copy(zdeg_v, deg_sh.at[pl.ds(base, NCH)])
        return 0
    lax.fori_loop(0, trip_n, _zero_spmem, 0)
    plsc.subcore_barrier()

    trip_d = (DEG_GROUPS - sid + NS - 1) // NS

    def _deg_group(k, _):
        row0 = (sid + NS * k) * KJ
        pltpu.sync_copy(src_hbm.at[pl.ds(row0, KJ)], idxs_v)
        pltpu.sync_copy(dst_hbm.at[pl.ds(row0, KJ)], idxd_v)
        descs = []
        for j in range(KJ):
            descs.append(pltpu.async_copy(
                ones_s_v, deg_sh.at[idxs_v.at[j]], sem_s, add=True))
            descs.append(pltpu.async_copy(
                ones_d_v, deg_sh.at[idxd_v.at[j]], sem_s, add=True))
        for d in descs:
            d.wait()
        return 0
    lax.fori_loop(0, trip_d, _deg_group, 0)
    plsc.subcore_barrier()

    def _node_chunk(k, _):
        base = (sid + NS * k) * NCH
        pltpu.sync_copy(deg_sh.at[pl.ds(base, NCH)], zdeg_v)
        for v in range(NCH // L):
            ridx = _iota16() + v * L
            do = plsc.load_gather(zdeg_v, [ridx, zi16])
            nsrc_v[pl.ds(k * NCH + v * L, L)] = _rsqrt16(jnp.maximum(do, 1.0))
            di = plsc.load_gather(zdeg_v, [ridx, zi16 + 1])
            ndst_v[pl.ds(k * NCH + v * L, L)] = _rsqrt16(jnp.maximum(di, 1.0))
        pltpu.sync_copy(x_hbm.at[pl.ds(base, NCH)], rows_a_v.at[pl.ds(0, NCH)])

        def _scale_grp(g, _):
            nv = nsrc_v[pl.ds(k * NCH + g * L, L)]
            for lane in range(L):
                s = nv[lane]
                r = g * L + lane
                for j in range(F // L):
                    rows_a_v[r, pl.ds(j * L, L)] = \
                        rows_a_v[r, pl.ds(j * L, L)] * s
            return 0
        lax.fori_loop(0, NCH // L, _scale_grp, 0)
        pltpu.sync_copy(rows_a_v.at[pl.ds(0, NCH)],
                        feat_hbm.at[pl.ds(base, NCH)])
        return 0
    lax.fori_loop(0, trip_n, _node_chunk, 0)
    plsc.subcore_barrier()

    trip_m = (MSG_GROUPS - sid + NS - 1) // NS
    erow0 = cid * (E // NC // CW)

    def _msg_group(k, _):
        row0 = erow0 + (sid + NS * k) * KJ
        pltpu.sync_copy(src_hbm.at[pl.ds(row0, KJ)], idxs_v)
        pltpu.sync_copy(dst_hbm.at[pl.ds(row0, KJ)], idxd_v)
        bufs = (rows_a_v, rows_b_v)
        gd = [None] * KJ
        sd = [None] * KJ
        gd[0] = pltpu.async_copy(feat_hbm.at[idxs_v.at[0]], bufs[0], sem_g)
        for j in range(KJ):
            cur = bufs[j % 2]
            gd[j].wait()
            if j + 1 < KJ:
                if j >= 1:
                    sd[j - 1].wait()
                gd[j + 1] = pltpu.async_copy(
                    feat_hbm.at[idxs_v.at[j + 1]], bufs[(j + 1) % 2], sem_g)
            sd[j] = pltpu.async_copy(
                cur, agg_sh.at[idxd_v.at[j]], sem_s, add=True)
        sd[KJ - 2].wait()
        sd[KJ - 1].wait()
        return 0
    lax.fori_loop(0, trip_m, _msg_group, 0)
    plsc.subcore_barrier()

    def _out_chunk(k, _):
        base = (sid + NS * k) * NCH
        pltpu.sync_copy(agg_sh.at[pl.ds(base, NCH)], rows_a_v.at[pl.ds(0, NCH)])

        def _scale_grp(g, _):
            nv = ndst_v[pl.ds(k * NCH + g * L, L)]
            for lane in range(L):
                s = nv[lane]
                r = g * L + lane
                for j in range(F // L):
                    rows_a_v[r, pl.ds(j * L, L)] = \
                        rows_a_v[r, pl.ds(j * L, L)] * s
            return 0
        lax.fori_loop(0, NCH // L, _scale_grp, 0)
        pltpu.sync_copy(rows_a_v.at[pl.ds(0, NCH)],
                        part_hbm.at[pl.ds(cid * N + base, NCH)])
        return 0
    lax.fori_loop(0, trip_n, _out_chunk, 0)


_sc_call = pl.kernel(
    _sc_body,
    out_type=(
        jax.ShapeDtypeStruct((NC * N, F), jnp.float32),
        jax.ShapeDtypeStruct((N, F), jnp.float32),
    ),
    mesh=plsc.VectorSubcoreMesh(core_axis_name="c", subcore_axis_name="s"),
    compiler_params=pltpu.CompilerParams(
        use_tc_tiling_on_sc=False, needs_layout_passes=False),
    scratch_types=[
        pltpu.VMEM_SHARED((N, F), jnp.float32),
        pltpu.VMEM_SHARED((N, DW), jnp.float32),
        pltpu.VMEM((KJ, CW), jnp.int32),
        pltpu.VMEM((KJ, CW), jnp.int32),
        pltpu.VMEM((CW, F), jnp.float32),
        pltpu.VMEM((CW, F), jnp.float32),
        pltpu.VMEM((CW, DW), jnp.float32),
        pltpu.VMEM((CW, DW), jnp.float32),
        pltpu.VMEM((NCH, DW), jnp.float32),
        pltpu.VMEM((NODE_CHUNKS // NS * NCH + NCH,), jnp.float32),
        pltpu.VMEM((NODE_CHUNKS // NS * NCH + NCH,), jnp.float32),
        pltpu.SemaphoreType.DMA,
        pltpu.SemaphoreType.DMA,
    ],
)


def _tc_body(p0_ref, p1_ref, w_ref, b_ref, o_ref):
    acc = p0_ref[...] + p1_ref[...]
    y = jnp.dot(acc, w_ref[...], preferred_element_type=jnp.float32,
                precision=lax.Precision.HIGHEST)
    o_ref[...] = jax.nn.sigmoid(y + b_ref[...])


@jax.jit
def kernel(x, edge_index, W, b):
    src2d = edge_index[0].reshape(EROWS, CW)
    dst2d = edge_index[1].reshape(EROWS, CW)
    part, _ = _sc_call(x, src2d, dst2d)
    out = pl.pallas_call(
        _tc_body,
        grid=(10,),
        in_specs=[
            pl.BlockSpec((N // 10, F), lambda i: (i, 0)),
            pl.BlockSpec((N // 10, F), lambda i: (i, 0)),
            pl.BlockSpec((F, F), lambda i: (0, 0)),
            pl.BlockSpec((1, F), lambda i: (0, 0)),
        ],
        out_specs=pl.BlockSpec((N // 10, F), lambda i: (i, 0)),
        out_shape=jax.ShapeDtypeStruct((N, F), jnp.float32),
    )(part[:N], part[N:], W, b.reshape(1, F))
    return out


# speedup vs baseline: 10.3112x; 1.0369x over previous
import jax
import jax.numpy as jnp
from jax import lax
from jax.experimental import pallas as pl
from jax.experimental.pallas import tpu as pltpu
from jax.experimental.pallas import tpu_sc as plsc

N = 10000
NP = 10240
E = 320000
EP = 327680
F = 128
NC = 2
NS = 16
L = 16

CW = 128
KJ = 10
EROWS = EP // CW
DEG_GP = 16
MSG_GP = 8
NCH = 80
NODE_GP = 8
DW = 8


def _iota16():
    return lax.iota(jnp.int32, 16)


def _rsqrt16(d):
    i = plsc.bitcast(d, jnp.int32)
    i = jnp.int32(0x5F3759DF) - lax.shift_right_logical(i, 1)
    y = plsc.bitcast(i, jnp.float32)
    for _ in range(3):
        y = y * (1.5 - 0.5 * d * y * y)
    return y


def _sc_body(x_hbm, src_hbm, dst_hbm, part_hbm, feat_hbm,
             agg_sh, deg_sh,
             idxs_v, idxd_v, rows_a_v, rows_b_v, ones_s_v, ones_d_v,
             zdeg_v, nsrc_v, ndst_v, sem_i, sem_g, sem_s):
    cid = lax.axis_index("c")
    sid = lax.axis_index("s")
    zf16 = jnp.zeros((16,), jnp.float32)
    zi16 = jnp.zeros((16,), jnp.int32)

    def _zero_row(r, _):
        for j in range(F // L):
            rows_a_v[r, pl.ds(j * L, L)] = zf16
        return 0
    lax.fori_loop(0, NCH, _zero_row, 0)

    def _init_narrow(g, _):
        p = _iota16() + g * L
        r = lax.shift_right_logical(p, 3)
        c = jnp.bitwise_and(p, 7)
        one_s = jnp.where(c == 0, 1.0, 0.0).astype(jnp.float32)
        one_d = jnp.where(c == 1, 1.0, 0.0).astype(jnp.float32)
        plsc.store_scatter(ones_s_v, [r, c], one_s)
        plsc.store_scatter(ones_d_v, [r, c], one_d)
        return 0
    lax.fori_loop(0, CW * DW // L, _init_narrow, 0)

    def _zero_narrow(g, _):
        p = _iota16() + g * L
        r = lax.shift_right_logical(p, 3)
        c = jnp.bitwise_and(p, 7)
        plsc.store_scatter(zdeg_v, [r, c], zf16)
        return 0
    lax.fori_loop(0, NCH * DW // L, _zero_narrow, 0)

    def _zero_spmem(k, _):
        base = (sid + NS * k) * NCH
        pltpu.sync_copy(rows_a_v.at[pl.ds(0, NCH)], agg_sh.at[pl.ds(base, NCH)])
        pltpu.sync_copy(zdeg_v, deg_sh.at[pl.ds(base, NCH)])
        return 0
    lax.fori_loop(0, NODE_GP, _zero_spmem, 0)
    plsc.subcore_barrier()

    def _deg_group(k, _):
        row0 = (sid + NS * k) * KJ
        di1 = pltpu.async_copy(src_hbm.at[pl.ds(row0, KJ)], idxs_v, sem_i)
        di2 = pltpu.async_copy(dst_hbm.at[pl.ds(row0, KJ)], idxd_v, sem_i)
        di1.wait()
        di2.wait()
        descs = []
        for j in range(KJ):
            descs.append(pltpu.async_copy(
                ones_s_v, deg_sh.at[idxs_v.at[j]], sem_s, add=True))
            descs.append(pltpu.async_copy(
                ones_d_v, deg_sh.at[idxd_v.at[j]], sem_s, add=True))
        for d in descs:
            d.wait()
        return 0
    lax.fori_loop(0, DEG_GP, _deg_group, 0)
    plsc.subcore_barrier()

    def _node_chunk(k, _):
        base = (sid + NS * k) * NCH
        pltpu.sync_copy(deg_sh.at[pl.ds(base, NCH)], zdeg_v)
        for v in range(NCH // L):
            ridx = _iota16() + v * L
            do = plsc.load_gather(zdeg_v, [ridx, zi16])
            nsrc_v[pl.ds(k * NCH + v * L, L)] = _rsqrt16(jnp.maximum(do, 1.0))
            di = plsc.load_gather(zdeg_v, [ridx, zi16 + 1])
            ndst_v[pl.ds(k * NCH + v * L, L)] = _rsqrt16(jnp.maximum(di, 1.0))
        pltpu.sync_copy(x_hbm.at[pl.ds(base, NCH)], rows_a_v.at[pl.ds(0, NCH)])

        def _scale_grp(g, _):
            nv = nsrc_v[pl.ds(k * NCH + g * L, L)]
            for lane in range(L):
                s = nv[lane]
                r = g * L + lane
                for j in range(F // L):
                    rows_a_v[r, pl.ds(j * L, L)] = \
                        rows_a_v[r, pl.ds(j * L, L)] * s
            return 0
        lax.fori_loop(0, NCH // L, _scale_grp, 0)
        pltpu.sync_copy(rows_a_v.at[pl.ds(0, NCH)],
                        feat_hbm.at[pl.ds(base, NCH)])
        return 0
    lax.fori_loop(0, NODE_GP, _node_chunk, 0)
    plsc.subcore_barrier()

    erow0 = cid * (EROWS // NC)

    def _msg_group(k, _):
        row0 = erow0 + (sid + NS * k) * KJ
        di1 = pltpu.async_copy(src_hbm.at[pl.ds(row0, KJ)], idxs_v, sem_i)
        di2 = pltpu.async_copy(dst_hbm.at[pl.ds(row0, KJ)], idxd_v, sem_i)
        di1.wait()
        di2.wait()
        bufs = (rows_a_v, rows_b_v)
        gd = [None] * KJ
        sd = [None] * KJ
        gd[0] = pltpu.async_copy(feat_hbm.at[idxs_v.at[0]], bufs[0], sem_g)
        for j in range(KJ):
            cur = bufs[j % 2]
            gd[j].wait()
            if j + 1 < KJ:
                if j >= 1:
                    sd[j - 1].wait()
                gd[j + 1] = pltpu.async_copy(
                    feat_hbm.at[idxs_v.at[j + 1]], bufs[(j + 1) % 2], sem_g)
            sd[j] = pltpu.async_copy(
                cur, agg_sh.at[idxd_v.at[j]], sem_s, add=True)
        sd[KJ - 2].wait()
        sd[KJ - 1].wait()
        return 0
    lax.fori_loop(0, MSG_GP, _msg_group, 0)
    plsc.subcore_barrier()

    def _out_chunk(k, _):
        base = (sid + NS * k) * NCH
        pltpu.sync_copy(agg_sh.at[pl.ds(base, NCH)], rows_a_v.at[pl.ds(0, NCH)])

        def _scale_grp(g, _):
            nv = ndst_v[pl.ds(k * NCH + g * L, L)]
            for lane in range(L):
                s = nv[lane]
                r = g * L + lane
                for j in range(F // L):
                    rows_a_v[r, pl.ds(j * L, L)] = \
                        rows_a_v[r, pl.ds(j * L, L)] * s
            return 0
        lax.fori_loop(0, NCH // L, _scale_grp, 0)
        pltpu.sync_copy(rows_a_v.at[pl.ds(0, NCH)],
                        part_hbm.at[pl.ds(cid * NP + base, NCH)])
        return 0
    lax.fori_loop(0, NODE_GP, _out_chunk, 0)


_sc_call = pl.kernel(
    _sc_body,
    out_type=(
        jax.ShapeDtypeStruct((NC * NP, F), jnp.float32),
        jax.ShapeDtypeStruct((NP, F), jnp.float32),
    ),
    mesh=plsc.VectorSubcoreMesh(core_axis_name="c", subcore_axis_name="s"),
    compiler_params=pltpu.CompilerParams(
        use_tc_tiling_on_sc=False, needs_layout_passes=False),
    scratch_types=[
        pltpu.VMEM_SHARED((NP, F), jnp.float32),
        pltpu.VMEM_SHARED((NP, DW), jnp.float32),
        pltpu.VMEM((KJ, CW), jnp.int32),
        pltpu.VMEM((KJ, CW), jnp.int32),
        pltpu.VMEM((CW, F), jnp.float32),
        pltpu.VMEM((CW, F), jnp.float32),
        pltpu.VMEM((CW, DW), jnp.float32),
        pltpu.VMEM((CW, DW), jnp.float32),
        pltpu.VMEM((NCH, DW), jnp.float32),
        pltpu.VMEM((NODE_GP * NCH,), jnp.float32),
        pltpu.VMEM((NODE_GP * NCH,), jnp.float32),
        pltpu.SemaphoreType.DMA,
        pltpu.SemaphoreType.DMA,
        pltpu.SemaphoreType.DMA,
    ],
)


def _tc_body(p0_ref, p1_ref, w_ref, b_ref, o_ref):
    acc = p0_ref[...] + p1_ref[...]
    y = jnp.dot(acc, w_ref[...], preferred_element_type=jnp.float32,
                precision=lax.Precision.HIGHEST)
    o_ref[...] = jax.nn.sigmoid(y + b_ref[...])


@jax.jit
def kernel(x, edge_index, W, b):
    xp = jnp.concatenate([x, jnp.zeros((NP - N, F), jnp.float32)], axis=0)
    pad = (jnp.arange(EP - E, dtype=jnp.int32) % (NP - N)) + N
    src2d = jnp.concatenate([edge_index[0], pad]).reshape(EROWS, CW)
    dst2d = jnp.concatenate([edge_index[1], pad]).reshape(EROWS, CW)
    part, _ = _sc_call(xp, src2d, dst2d)
    out = pl.pallas_call(
        _tc_body,
        grid=(10,),
        in_specs=[
            pl.BlockSpec((N // 10, F), lambda i: (i, 0)),
            pl.BlockSpec((N // 10, F), lambda i: (i, 0)),
            pl.BlockSpec((F, F), lambda i: (0, 0)),
            pl.BlockSpec((1, F), lambda i: (0, 0)),
        ],
        out_specs=pl.BlockSpec((N // 10, F), lambda i: (i, 0)),
        out_shape=jax.ShapeDtypeStruct((N, F), jnp.float32),
    )(part[:N], part[NP:NP + N], W, b.reshape(1, F))
    return out


# speedup vs baseline: 10.6114x; 1.0291x over previous
import jax
import jax.numpy as jnp
from jax import lax
from jax.experimental import pallas as pl
from jax.experimental.pallas import tpu as pltpu
from jax.experimental.pallas import tpu_sc as plsc

N = 10000
NP = 10240
E = 320000
EP = 327680
F = 128
NC = 2
NS = 16
L = 16

CW = 128
KJ = 10
EROWS = EP // CW
DEG_GP = 16
MSG_GP = 8
NCH = 80
NODE_GP = 8
DW = 8


def _iota16():
    return lax.iota(jnp.int32, 16)


def _rsqrt16(d):
    i = plsc.bitcast(d, jnp.int32)
    i = jnp.int32(0x5F3759DF) - lax.shift_right_logical(i, 1)
    y = plsc.bitcast(i, jnp.float32)
    for _ in range(3):
        y = y * (1.5 - 0.5 * d * y * y)
    return y


def _sc_body(x_hbm, src_hbm, dst_hbm, part_hbm, feat_hbm,
             agg_sh, deg_sh,
             idxs_v, idxd_v, idxs_b, idxd_b, rows_a_v, rows_b_v,
             ones_s_v, ones_d_v, zdeg_v, nsrc_v, ndst_v,
             sem_i, sem_g, sem_s):
    cid = lax.axis_index("c")
    sid = lax.axis_index("s")
    zf16 = jnp.zeros((16,), jnp.float32)
    zi16 = jnp.zeros((16,), jnp.int32)

    def _zero_row(r, _):
        for j in range(F // L):
            rows_a_v[r, pl.ds(j * L, L)] = zf16
        return 0
    lax.fori_loop(0, NCH, _zero_row, 0)

    def _init_narrow(g, _):
        p = _iota16() + g * L
        r = lax.shift_right_logical(p, 3)
        c = jnp.bitwise_and(p, 7)
        one_s = jnp.where(c == 0, 1.0, 0.0).astype(jnp.float32)
        one_d = jnp.where(c == 1, 1.0, 0.0).astype(jnp.float32)
        plsc.store_scatter(ones_s_v, [r, c], one_s)
        plsc.store_scatter(ones_d_v, [r, c], one_d)
        return 0
    lax.fori_loop(0, CW * DW // L, _init_narrow, 0)

    def _zero_narrow(g, _):
        p = _iota16() + g * L
        r = lax.shift_right_logical(p, 3)
        c = jnp.bitwise_and(p, 7)
        plsc.store_scatter(zdeg_v, [r, c], zf16)
        return 0
    lax.fori_loop(0, NCH * DW // L, _zero_narrow, 0)

    def _zero_spmem(k, _):
        base = (sid + NS * k) * NCH
        pltpu.sync_copy(rows_a_v.at[pl.ds(0, NCH)], agg_sh.at[pl.ds(base, NCH)])
        pltpu.sync_copy(zdeg_v, deg_sh.at[pl.ds(base, NCH)])
        return 0
    lax.fori_loop(0, NODE_GP, _zero_spmem, 0)
    plsc.subcore_barrier()

    def _deg_group(k, _):
        row0 = (sid + NS * k) * KJ
        di1 = pltpu.async_copy(src_hbm.at[pl.ds(row0, KJ)], idxs_v, sem_i)
        di2 = pltpu.async_copy(dst_hbm.at[pl.ds(row0, KJ)], idxd_v, sem_i)
        di1.wait()
        di2.wait()
        descs = []
        for j in range(KJ):
            descs.append(pltpu.async_copy(
                ones_s_v, deg_sh.at[idxs_v.at[j]], sem_s, add=True))
            descs.append(pltpu.async_copy(
                ones_d_v, deg_sh.at[idxd_v.at[j]], sem_s, add=True))
        for d in descs:
            d.wait()
        return 0
    lax.fori_loop(0, DEG_GP, _deg_group, 0)
    plsc.subcore_barrier()

    def _node_chunk(k, _):
        base = (sid + NS * k) * NCH
        pltpu.sync_copy(deg_sh.at[pl.ds(base, NCH)], zdeg_v)
        for v in range(NCH // L):
            ridx = _iota16() + v * L
            do = plsc.load_gather(zdeg_v, [ridx, zi16])
            nsrc_v[pl.ds(k * NCH + v * L, L)] = _rsqrt16(jnp.maximum(do, 1.0))
            di = plsc.load_gather(zdeg_v, [ridx, zi16 + 1])
            ndst_v[pl.ds(k * NCH + v * L, L)] = _rsqrt16(jnp.maximum(di, 1.0))
        pltpu.sync_copy(x_hbm.at[pl.ds(base, NCH)], rows_a_v.at[pl.ds(0, NCH)])

        def _scale_grp(g, _):
            nv = nsrc_v[pl.ds(k * NCH + g * L, L)]
            for lane in range(L):
                s = nv[lane]
                r = g * L + lane
                for j in range(F // L):
                    rows_a_v[r, pl.ds(j * L, L)] = \
                        rows_a_v[r, pl.ds(j * L, L)] * s
            return 0
        lax.fori_loop(0, NCH // L, _scale_grp, 0)
        pltpu.sync_copy(rows_a_v.at[pl.ds(0, NCH)],
                        feat_hbm.at[pl.ds(base, NCH)])
        return 0
    lax.fori_loop(0, NODE_GP, _node_chunk, 0)
    plsc.subcore_barrier()

    erow0 = cid * (EROWS // NC)
    ipair = ((idxs_v, idxd_v), (idxs_b, idxd_b))
    rbuf = (rows_a_v, rows_b_v)

    def _msg_idx(m):
        row0 = erow0 + (sid + NS * m) * KJ
        pa = ipair[m % 2]
        return [pltpu.async_copy(src_hbm.at[pl.ds(row0, KJ)], pa[0], sem_i),
                pltpu.async_copy(dst_hbm.at[pl.ds(row0, KJ)], pa[1], sem_i)]

    S = MSG_GP * KJ
    gd = {}
    sd = {}
    msg_id = {0: _msg_idx(0)}
    for d in msg_id[0]:
        d.wait()
    msg_id[1] = _msg_idx(1)
    gd[0] = pltpu.async_copy(feat_hbm.at[ipair[0][0].at[0]], rbuf[0], sem_g)
    for s in range(S):
        m, j = divmod(s, KJ)
        cur = rbuf[s % 2]
        gd[s].wait()
        if s + 1 < S:
            m1, j1 = divmod(s + 1, KJ)
            if s >= 1:
                sd[s - 1].wait()
            if j1 == 0:
                for d in msg_id[m1]:
                    d.wait()
            gd[s + 1] = pltpu.async_copy(
                feat_hbm.at[ipair[m1 % 2][0].at[j1]],
                rbuf[(s + 1) % 2], sem_g)
        sd[s] = pltpu.async_copy(cur, agg_sh.at[ipair[m % 2][1].at[j]],
                                 sem_s, add=True)
        if j == 2 and 2 <= m + 1 < MSG_GP:
            msg_id[m + 1] = _msg_idx(m + 1)
    sd[S - 2].wait()
    sd[S - 1].wait()
    plsc.subcore_barrier()

    def _out_chunk(k, _):
        base = (sid + NS * k) * NCH
        pltpu.sync_copy(agg_sh.at[pl.ds(base, NCH)], rows_a_v.at[pl.ds(0, NCH)])

        def _scale_grp(g, _):
            nv = ndst_v[pl.ds(k * NCH + g * L, L)]
            for lane in range(L):
                s = nv[lane]
                r = g * L + lane
                for j in range(F // L):
                    rows_a_v[r, pl.ds(j * L, L)] = \
                        rows_a_v[r, pl.ds(j * L, L)] * s
            return 0
        lax.fori_loop(0, NCH // L, _scale_grp, 0)
        pltpu.sync_copy(rows_a_v.at[pl.ds(0, NCH)],
                        part_hbm.at[pl.ds(cid * NP + base, NCH)])
        return 0
    lax.fori_loop(0, NODE_GP, _out_chunk, 0)


_sc_call = pl.kernel(
    _sc_body,
    out_type=(
        jax.ShapeDtypeStruct((NC * NP, F), jnp.float32),
        jax.ShapeDtypeStruct((NP, F), jnp.float32),
    ),
    mesh=plsc.VectorSubcoreMesh(core_axis_name="c", subcore_axis_name="s"),
    compiler_params=pltpu.CompilerParams(
        use_tc_tiling_on_sc=False, needs_layout_passes=False),
    scratch_types=[
        pltpu.VMEM_SHARED((NP, F), jnp.float32),
        pltpu.VMEM_SHARED((NP, DW), jnp.float32),
        pltpu.VMEM((KJ, CW), jnp.int32),
        pltpu.VMEM((KJ, CW), jnp.int32),
        pltpu.VMEM((KJ, CW), jnp.int32),
        pltpu.VMEM((KJ, CW), jnp.int32),
        pltpu.VMEM((CW, F), jnp.float32),
        pltpu.VMEM((CW, F), jnp.float32),
        pltpu.VMEM((CW, DW), jnp.float32),
        pltpu.VMEM((CW, DW), jnp.float32),
        pltpu.VMEM((NCH, DW), jnp.float32),
        pltpu.VMEM((NODE_GP * NCH,), jnp.float32),
        pltpu.VMEM((NODE_GP * NCH,), jnp.float32),
        pltpu.SemaphoreType.DMA,
        pltpu.SemaphoreType.DMA,
        pltpu.SemaphoreType.DMA,
    ],
)


def _tc_body(p0_ref, p1_ref, w_ref, b_ref, o_ref):
    acc = p0_ref[...] + p1_ref[...]
    y = jnp.dot(acc, w_ref[...], preferred_element_type=jnp.float32,
                precision=lax.Precision.HIGHEST)
    o_ref[...] = jax.nn.sigmoid(y + b_ref[...])


@jax.jit
def kernel(x, edge_index, W, b):
    xp = jnp.concatenate([x, jnp.zeros((NP - N, F), jnp.float32)], axis=0)
    pad = (jnp.arange(EP - E, dtype=jnp.int32) % (NP - N)) + N
    src2d = jnp.concatenate([edge_index[0], pad]).reshape(EROWS, CW)
    dst2d = jnp.concatenate([edge_index[1], pad]).reshape(EROWS, CW)
    part, _ = _sc_call(xp, src2d, dst2d)
    out = pl.pallas_call(
        _tc_body,
        grid=(10,),
        in_specs=[
            pl.BlockSpec((N // 10, F), lambda i: (i, 0)),
            pl.BlockSpec((N // 10, F), lambda i: (i, 0)),
            pl.BlockSpec((F, F), lambda i: (0, 0)),
            pl.BlockSpec((1, F), lambda i: (0, 0)),
        ],
        out_specs=pl.BlockSpec((N // 10, F), lambda i: (i, 0)),
        out_shape=jax.ShapeDtypeStruct((N, F), jnp.float32),
    )(part[:N], part[NP:NP + N], W, b.reshape(1, F))
    return out


# speedup vs baseline: 10.6263x; 1.0014x over previous
import jax
import jax.numpy as jnp
from jax import lax
from jax.experimental import pallas as pl
from jax.experimental.pallas import tpu as pltpu
from jax.experimental.pallas import tpu_sc as plsc

N = 10000
NP = 10240
E = 320000
EP = 327680
F = 128
NC = 2
NS = 16
L = 16

CW = 128
KJ = 10
EROWS = EP // CW
DEG_GP = 16
MSG_GP = 8
NCH = 80
NODE_GP = 8
DW = 8


def _iota16():
    return lax.iota(jnp.int32, 16)


def _rsqrt16(d):
    i = plsc.bitcast(d, jnp.int32)
    i = jnp.int32(0x5F3759DF) - lax.shift_right_logical(i, 1)
    y = plsc.bitcast(i, jnp.float32)
    for _ in range(3):
        y = y * (1.5 - 0.5 * d * y * y)
    return y


def _sc_body(x_hbm, src_hbm, dst_hbm, part_hbm, feat_hbm,
             agg_sh, deg_sh,
             idxs_v, idxd_v, idxs_b, idxd_b, rows_a_v, rows_b_v,
             ones_s_v, ones_d_v, zdeg_v, nsrc_v, ndst_v,
             sem_i, sem_g, sem_s):
    cid = lax.axis_index("c")
    sid = lax.axis_index("s")
    zf16 = jnp.zeros((16,), jnp.float32)
    zi16 = jnp.zeros((16,), jnp.int32)

    _sc0 = jax.named_scope("ph0_zero"); _sc0.__enter__()
    def _zero_row(r, _):
        for j in range(F // L):
            rows_a_v[r, pl.ds(j * L, L)] = zf16
        return 0
    lax.fori_loop(0, NCH, _zero_row, 0)

    def _init_narrow(g, _):
        p = _iota16() + g * L
        r = lax.shift_right_logical(p, 3)
        c = jnp.bitwise_and(p, 7)
        one_s = jnp.where(c == 0, 1.0, 0.0).astype(jnp.float32)
        one_d = jnp.where(c == 1, 1.0, 0.0).astype(jnp.float32)
        plsc.store_scatter(ones_s_v, [r, c], one_s)
        plsc.store_scatter(ones_d_v, [r, c], one_d)
        return 0
    lax.fori_loop(0, CW * DW // L, _init_narrow, 0)

    def _zero_narrow(g, _):
        p = _iota16() + g * L
        r = lax.shift_right_logical(p, 3)
        c = jnp.bitwise_and(p, 7)
        plsc.store_scatter(zdeg_v, [r, c], zf16)
        return 0
    lax.fori_loop(0, NCH * DW // L, _zero_narrow, 0)

    def _zero_spmem(k, _):
        base = (sid + NS * k) * NCH
        pltpu.sync_copy(rows_a_v.at[pl.ds(0, NCH)], agg_sh.at[pl.ds(base, NCH)])
        pltpu.sync_copy(zdeg_v, deg_sh.at[pl.ds(base, NCH)])
        return 0
    lax.fori_loop(0, NODE_GP, _zero_spmem, 0)
    plsc.subcore_barrier()
    _sc0.__exit__(None, None, None)

    _sc1 = jax.named_scope("ph1_deg"); _sc1.__enter__()
    def _deg_group(k, _):
        row0 = (sid + NS * k) * KJ
        di1 = pltpu.async_copy(src_hbm.at[pl.ds(row0, KJ)], idxs_v, sem_i)
        di2 = pltpu.async_copy(dst_hbm.at[pl.ds(row0, KJ)], idxd_v, sem_i)
        di1.wait()
        di2.wait()
        descs = []
        for j in range(KJ):
            descs.append(pltpu.async_copy(
                ones_s_v, deg_sh.at[idxs_v.at[j]], sem_s, add=True))
            descs.append(pltpu.async_copy(
                ones_d_v, deg_sh.at[idxd_v.at[j]], sem_s, add=True))
        for d in descs:
            d.wait()
        return 0
    lax.fori_loop(0, DEG_GP, _deg_group, 0)
    plsc.subcore_barrier()
    _sc1.__exit__(None, None, None)

    _sc2 = jax.named_scope("ph2_feat"); _sc2.__enter__()
    def _node_chunk(k, _):
        base = (sid + NS * k) * NCH
        pltpu.sync_copy(deg_sh.at[pl.ds(base, NCH)], zdeg_v)
        for v in range(NCH // L):
            ridx = _iota16() + v * L
            do = plsc.load_gather(zdeg_v, [ridx, zi16])
            nsrc_v[pl.ds(k * NCH + v * L, L)] = _rsqrt16(jnp.maximum(do, 1.0))
            di = plsc.load_gather(zdeg_v, [ridx, zi16 + 1])
            ndst_v[pl.ds(k * NCH + v * L, L)] = _rsqrt16(jnp.maximum(di, 1.0))
        pltpu.sync_copy(x_hbm.at[pl.ds(base, NCH)], rows_a_v.at[pl.ds(0, NCH)])

        def _scale_grp(g, _):
            nv = nsrc_v[pl.ds(k * NCH + g * L, L)]
            for lane in range(L):
                s = nv[lane]
                r = g * L + lane
                for j in range(F // L):
                    rows_a_v[r, pl.ds(j * L, L)] = \
                        rows_a_v[r, pl.ds(j * L, L)] * s
            return 0
        lax.fori_loop(0, NCH // L, _scale_grp, 0)
        pltpu.sync_copy(rows_a_v.at[pl.ds(0, NCH)],
                        feat_hbm.at[pl.ds(base, NCH)])
        return 0
    lax.fori_loop(0, NODE_GP, _node_chunk, 0)
    plsc.subcore_barrier()
    _sc2.__exit__(None, None, None)

    _sc3 = jax.named_scope("ph3_msg"); _sc3.__enter__()
    erow0 = cid * (EROWS // NC)
    ipair = ((idxs_v, idxd_v), (idxs_b, idxd_b))
    rbuf = (rows_a_v, rows_b_v)

    def _msg_idx(m):
        row0 = erow0 + (sid + NS * m) * KJ
        pa = ipair[m % 2]
        return [pltpu.async_copy(src_hbm.at[pl.ds(row0, KJ)], pa[0], sem_i),
                pltpu.async_copy(dst_hbm.at[pl.ds(row0, KJ)], pa[1], sem_i)]

    S = MSG_GP * KJ
    gd = {}
    sd = {}
    msg_id = {0: _msg_idx(0)}
    for d in msg_id[0]:
        d.wait()
    msg_id[1] = _msg_idx(1)
    gd[0] = pltpu.async_copy(feat_hbm.at[ipair[0][0].at[0]], rbuf[0], sem_g)
    for s in range(S):
        m, j = divmod(s, KJ)
        cur = rbuf[s % 2]
        gd[s].wait()
        if s + 1 < S:
            m1, j1 = divmod(s + 1, KJ)
            if s >= 1:
                sd[s - 1].wait()
            if j1 == 0:
                for d in msg_id[m1]:
                    d.wait()
            gd[s + 1] = pltpu.async_copy(
                feat_hbm.at[ipair[m1 % 2][0].at[j1]],
                rbuf[(s + 1) % 2], sem_g)
        sd[s] = pltpu.async_copy(cur, agg_sh.at[ipair[m % 2][1].at[j]],
                                 sem_s, add=True)
        if j == 2 and 2 <= m + 1 < MSG_GP:
            msg_id[m + 1] = _msg_idx(m + 1)
    sd[S - 2].wait()
    sd[S - 1].wait()
    plsc.subcore_barrier()
    _sc3.__exit__(None, None, None)

    _sc4 = jax.named_scope("ph4_out"); _sc4.__enter__()
    def _out_chunk(k, _):
        base = (sid + NS * k) * NCH
        pltpu.sync_copy(agg_sh.at[pl.ds(base, NCH)], rows_a_v.at[pl.ds(0, NCH)])

        def _scale_grp(g, _):
            nv = ndst_v[pl.ds(k * NCH + g * L, L)]
            for lane in range(L):
                s = nv[lane]
                r = g * L + lane
                for j in range(F // L):
                    rows_a_v[r, pl.ds(j * L, L)] = \
                        rows_a_v[r, pl.ds(j * L, L)] * s
            return 0
        lax.fori_loop(0, NCH // L, _scale_grp, 0)
        pltpu.sync_copy(rows_a_v.at[pl.ds(0, NCH)],
                        part_hbm.at[pl.ds(cid * NP + base, NCH)])
        return 0
    lax.fori_loop(0, NODE_GP, _out_chunk, 0)
    _sc4.__exit__(None, None, None)


_sc_call = pl.kernel(
    _sc_body,
    out_type=(
        jax.ShapeDtypeStruct((NC * NP, F), jnp.float32),
        jax.ShapeDtypeStruct((NP, F), jnp.float32),
    ),
    mesh=plsc.VectorSubcoreMesh(core_axis_name="c", subcore_axis_name="s"),
    compiler_params=pltpu.CompilerParams(
        use_tc_tiling_on_sc=False, needs_layout_passes=False),
    scratch_types=[
        pltpu.VMEM_SHARED((NP, F), jnp.float32),
        pltpu.VMEM_SHARED((NP, DW), jnp.float32),
        pltpu.VMEM((KJ, CW), jnp.int32),
        pltpu.VMEM((KJ, CW), jnp.int32),
        pltpu.VMEM((KJ, CW), jnp.int32),
        pltpu.VMEM((KJ, CW), jnp.int32),
        pltpu.VMEM((CW, F), jnp.float32),
        pltpu.VMEM((CW, F), jnp.float32),
        pltpu.VMEM((CW, DW), jnp.float32),
        pltpu.VMEM((CW, DW), jnp.float32),
        pltpu.VMEM((NCH, DW), jnp.float32),
        pltpu.VMEM((NODE_GP * NCH,), jnp.float32),
        pltpu.VMEM((NODE_GP * NCH,), jnp.float32),
        pltpu.SemaphoreType.DMA,
        pltpu.SemaphoreType.DMA,
        pltpu.SemaphoreType.DMA,
    ],
)


def _tc_body(p0_ref, p1_ref, w_ref, b_ref, o_ref):
    acc = p0_ref[...] + p1_ref[...]
    y = jnp.dot(acc, w_ref[...], preferred_element_type=jnp.float32,
                precision=lax.Precision.HIGHEST)
    o_ref[...] = jax.nn.sigmoid(y + b_ref[...])


@jax.jit
def kernel(x, edge_index, W, b):
    xp = jnp.concatenate([x, jnp.zeros((NP - N, F), jnp.float32)], axis=0)
    pad = (jnp.arange(EP - E, dtype=jnp.int32) % (NP - N)) + N
    src2d = jnp.concatenate([edge_index[0], pad]).reshape(EROWS, CW)
    dst2d = jnp.concatenate([edge_index[1], pad]).reshape(EROWS, CW)
    part, _ = _sc_call(xp, src2d, dst2d)
    out = pl.pallas_call(
        _tc_body,
        grid=(10,),
        in_specs=[
            pl.BlockSpec((N // 10, F), lambda i: (i, 0)),
            pl.BlockSpec((N // 10, F), lambda i: (i, 0)),
            pl.BlockSpec((F, F), lambda i: (0, 0)),
            pl.BlockSpec((1, F), lambda i: (0, 0)),
        ],
        out_specs=pl.BlockSpec((N // 10, F), lambda i: (i, 0)),
        out_shape=jax.ShapeDtypeStruct((N, F), jnp.float32),
    )(part[:N], part[NP:NP + N], W, b.reshape(1, F))
    return out


# speedup vs baseline: 10.7582x; 1.0124x over previous
import jax
import jax.numpy as jnp
from jax import lax
from jax.experimental import pallas as pl
from jax.experimental.pallas import tpu as pltpu
from jax.experimental.pallas import tpu_sc as plsc

N = 10000
NP = 10240
E = 320000
EP = 327680
F = 128
NC = 2
NS = 16
L = 16

CW = 128
KJ = 10
EROWS = EP // CW
DEG_GP = 16
MSG_GP = 8
NCH = 80
NODE_GP = 8
DW = 8


def _iota16():
    return lax.iota(jnp.int32, 16)


def _rsqrt16(d):
    i = plsc.bitcast(d, jnp.int32)
    i = jnp.int32(0x5F3759DF) - lax.shift_right_logical(i, 1)
    y = plsc.bitcast(i, jnp.float32)
    for _ in range(3):
        y = y * (1.5 - 0.5 * d * y * y)
    return y


def _sc_body(x_hbm, src_hbm, dst_hbm, part_hbm, feat_hbm,
             agg_sh, deg_sh,
             idxs_v, idxd_v, idxs_b, idxd_b, rows_a_v, rows_b_v,
             ones_s_v, ones_d_v, zdeg_v, nsrc_v, ndst_v,
             sem_i, sem_g, sem_s):
    cid = lax.axis_index("c")
    sid = lax.axis_index("s")
    zf16 = jnp.zeros((16,), jnp.float32)
    zi16 = jnp.zeros((16,), jnp.int32)

    _sc0 = jax.named_scope("ph0_zero"); _sc0.__enter__()
    def _zero_row(r, _):
        for j in range(F // L):
            rows_a_v[r, pl.ds(j * L, L)] = zf16
        return 0
    lax.fori_loop(0, NCH, _zero_row, 0)

    def _init_narrow(g, _):
        p = _iota16() + g * L
        r = lax.shift_right_logical(p, 3)
        c = jnp.bitwise_and(p, 7)
        one_s = jnp.where(c == 0, 1.0, 0.0).astype(jnp.float32)
        one_d = jnp.where(c == 1, 1.0, 0.0).astype(jnp.float32)
        plsc.store_scatter(ones_s_v, [r, c], one_s)
        plsc.store_scatter(ones_d_v, [r, c], one_d)
        return 0
    lax.fori_loop(0, CW * DW // L, _init_narrow, 0)

    def _zero_narrow(g, _):
        p = _iota16() + g * L
        r = lax.shift_right_logical(p, 3)
        c = jnp.bitwise_and(p, 7)
        plsc.store_scatter(zdeg_v, [r, c], zf16)
        return 0
    lax.fori_loop(0, NCH * DW // L, _zero_narrow, 0)

    def _zero_spmem(k, _):
        base = (sid + NS * k) * NCH
        pltpu.sync_copy(rows_a_v.at[pl.ds(0, NCH)], agg_sh.at[pl.ds(base, NCH)])
        pltpu.sync_copy(zdeg_v, deg_sh.at[pl.ds(base, NCH)])
        return 0
    lax.fori_loop(0, NODE_GP, _zero_spmem, 0)
    plsc.subcore_barrier()
    _sc0.__exit__(None, None, None)

    _sc1 = jax.named_scope("ph1_deg"); _sc1.__enter__()
    def _deg_group(k, _):
        row0 = (sid + NS * k) * KJ
        di1 = pltpu.async_copy(src_hbm.at[pl.ds(row0, KJ)], idxs_v, sem_i)
        di2 = pltpu.async_copy(dst_hbm.at[pl.ds(row0, KJ)], idxd_v, sem_i)
        di1.wait()
        di2.wait()
        descs = []
        for j in range(KJ):
            descs.append(pltpu.async_copy(
                ones_s_v, deg_sh.at[idxs_v.at[j]], sem_s, add=True))
            descs.append(pltpu.async_copy(
                ones_d_v, deg_sh.at[idxd_v.at[j]], sem_s, add=True))
        for d in descs:
            d.wait()
        return 0
    lax.fori_loop(0, DEG_GP, _deg_group, 0)
    plsc.subcore_barrier()
    _sc1.__exit__(None, None, None)

    _sc2 = jax.named_scope("ph2_feat"); _sc2.__enter__()
    def _node_chunk(k, _):
        base = (sid + NS * k) * NCH
        pltpu.sync_copy(deg_sh.at[pl.ds(base, NCH)], zdeg_v)
        for v in range(NCH // L):
            ridx = _iota16() + v * L
            do = plsc.load_gather(zdeg_v, [ridx, zi16])
            nsrc_v[pl.ds(k * NCH + v * L, L)] = _rsqrt16(jnp.maximum(do, 1.0))
            di = plsc.load_gather(zdeg_v, [ridx, zi16 + 1])
            ndst_v[pl.ds(k * NCH + v * L, L)] = _rsqrt16(jnp.maximum(di, 1.0))
        xbase = jnp.minimum(base, N - NCH)
        pltpu.sync_copy(x_hbm.at[pl.ds(xbase, NCH)], rows_a_v.at[pl.ds(0, NCH)])

        def _scale_grp(g, _):
            nv = nsrc_v[pl.ds(k * NCH + g * L, L)]
            for lane in range(L):
                s = nv[lane]
                r = g * L + lane
                for j in range(F // L):
                    rows_a_v[r, pl.ds(j * L, L)] = \
                        rows_a_v[r, pl.ds(j * L, L)] * s
            return 0
        lax.fori_loop(0, NCH // L, _scale_grp, 0)
        pltpu.sync_copy(rows_a_v.at[pl.ds(0, NCH)],
                        feat_hbm.at[pl.ds(base, NCH)])
        return 0
    lax.fori_loop(0, NODE_GP, _node_chunk, 0)
    plsc.subcore_barrier()
    _sc2.__exit__(None, None, None)

    _sc3 = jax.named_scope("ph3_msg"); _sc3.__enter__()
    erow0 = cid * (EROWS // NC)
    ipair = ((idxs_v, idxd_v), (idxs_b, idxd_b))
    rbuf = (rows_a_v, rows_b_v)

    def _msg_idx(m):
        row0 = erow0 + (sid + NS * m) * KJ
        pa = ipair[m % 2]
        return [pltpu.async_copy(src_hbm.at[pl.ds(row0, KJ)], pa[0], sem_i),
                pltpu.async_copy(dst_hbm.at[pl.ds(row0, KJ)], pa[1], sem_i)]

    S = MSG_GP * KJ
    gd = {}
    sd = {}
    msg_id = {0: _msg_idx(0)}
    for d in msg_id[0]:
        d.wait()
    msg_id[1] = _msg_idx(1)
    gd[0] = pltpu.async_copy(feat_hbm.at[ipair[0][0].at[0]], rbuf[0], sem_g)
    for s in range(S):
        m, j = divmod(s, KJ)
        cur = rbuf[s % 2]
        gd[s].wait()
        if s + 1 < S:
            m1, j1 = divmod(s + 1, KJ)
            if s >= 1:
                sd[s - 1].wait()
            if j1 == 0:
                for d in msg_id[m1]:
                    d.wait()
            gd[s + 1] = pltpu.async_copy(
                feat_hbm.at[ipair[m1 % 2][0].at[j1]],
                rbuf[(s + 1) % 2], sem_g)
        sd[s] = pltpu.async_copy(cur, agg_sh.at[ipair[m % 2][1].at[j]],
                                 sem_s, add=True)
        if j == 2 and 2 <= m + 1 < MSG_GP:
            msg_id[m + 1] = _msg_idx(m + 1)
    sd[S - 2].wait()
    sd[S - 1].wait()
    plsc.subcore_barrier()
    _sc3.__exit__(None, None, None)

    _sc4 = jax.named_scope("ph4_out"); _sc4.__enter__()
    def _out_chunk(k, _):
        base = (sid + NS * k) * NCH
        pltpu.sync_copy(agg_sh.at[pl.ds(base, NCH)], rows_a_v.at[pl.ds(0, NCH)])

        def _scale_grp(g, _):
            nv = ndst_v[pl.ds(k * NCH + g * L, L)]
            for lane in range(L):
                s = nv[lane]
                r = g * L + lane
                for j in range(F // L):
                    rows_a_v[r, pl.ds(j * L, L)] = \
                        rows_a_v[r, pl.ds(j * L, L)] * s
            return 0
        lax.fori_loop(0, NCH // L, _scale_grp, 0)
        pltpu.sync_copy(rows_a_v.at[pl.ds(0, NCH)],
                        part_hbm.at[pl.ds(cid * NP + base, NCH)])
        return 0
    lax.fori_loop(0, NODE_GP, _out_chunk, 0)
    _sc4.__exit__(None, None, None)


_sc_call = pl.kernel(
    _sc_body,
    out_type=(
        jax.ShapeDtypeStruct((NC * NP, F), jnp.float32),
        jax.ShapeDtypeStruct((NP, F), jnp.float32),
    ),
    mesh=plsc.VectorSubcoreMesh(core_axis_name="c", subcore_axis_name="s"),
    compiler_params=pltpu.CompilerParams(
        use_tc_tiling_on_sc=False, needs_layout_passes=False),
    scratch_types=[
        pltpu.VMEM_SHARED((NP, F), jnp.float32),
        pltpu.VMEM_SHARED((NP, DW), jnp.float32),
        pltpu.VMEM((KJ, CW), jnp.int32),
        pltpu.VMEM((KJ, CW), jnp.int32),
        pltpu.VMEM((KJ, CW), jnp.int32),
        pltpu.VMEM((KJ, CW), jnp.int32),
        pltpu.VMEM((CW, F), jnp.float32),
        pltpu.VMEM((CW, F), jnp.float32),
        pltpu.VMEM((CW, DW), jnp.float32),
        pltpu.VMEM((CW, DW), jnp.float32),
        pltpu.VMEM((NCH, DW), jnp.float32),
        pltpu.VMEM((NODE_GP * NCH,), jnp.float32),
        pltpu.VMEM((NODE_GP * NCH,), jnp.float32),
        pltpu.SemaphoreType.DMA,
        pltpu.SemaphoreType.DMA,
        pltpu.SemaphoreType.DMA,
    ],
)


def _tc_body(p0_ref, p1_ref, w_ref, b_ref, o_ref):
    acc = p0_ref[...] + p1_ref[...]
    y = jnp.dot(acc, w_ref[...], preferred_element_type=jnp.float32,
                precision=lax.Precision.HIGHEST)
    o_ref[...] = jax.nn.sigmoid(y + b_ref[...])


@jax.jit
def kernel(x, edge_index, W, b):
    pad = (jnp.arange(EP - E, dtype=jnp.int32) % (NP - N)) + N
    src2d = jnp.concatenate([edge_index[0], pad]).reshape(EROWS, CW)
    dst2d = jnp.concatenate([edge_index[1], pad]).reshape(EROWS, CW)
    part, _ = _sc_call(x, src2d, dst2d)
    out = pl.pallas_call(
        _tc_body,
        grid=(10,),
        in_specs=[
            pl.BlockSpec((N // 10, F), lambda i: (i, 0)),
            pl.BlockSpec((N // 10, F), lambda i: (i, 0)),
            pl.BlockSpec((F, F), lambda i: (0, 0)),
            pl.BlockSpec((1, F), lambda i: (0, 0)),
        ],
        out_specs=pl.BlockSpec((N // 10, F), lambda i: (i, 0)),
        out_shape=jax.ShapeDtypeStruct((N, F), jnp.float32),
    )(part[:N], part[NP:NP + N], W, b.reshape(1, F))
    return out


# speedup vs baseline: 11.3573x; 1.0557x over previous
import jax
import jax.numpy as jnp
from jax import lax
from jax.experimental import pallas as pl
from jax.experimental.pallas import tpu as pltpu
from jax.experimental.pallas import tpu_sc as plsc

N = 10000
NP = 10240
E = 320000
EP = 327680
F = 128
NC = 2
NS = 16
L = 16

CW = 128
KJ = 10
EROWS = EP // CW
DEG_GP = 16
CW3 = 64
EROWS3 = EP // CW3
MSG_GP = 16
NCH = 64
NODE_GP = 10
DW = 8


def _iota16():
    return lax.iota(jnp.int32, 16)


def _rsqrt16(d):
    i = plsc.bitcast(d, jnp.int32)
    i = jnp.int32(0x5F3759DF) - lax.shift_right_logical(i, 1)
    y = plsc.bitcast(i, jnp.float32)
    for _ in range(3):
        y = y * (1.5 - 0.5 * d * y * y)
    return y


def _sc_body(x_hbm, src_hbm, dst_hbm, src3_hbm, dst3_hbm,
             part_hbm, feat_hbm,
             agg_sh, deg_sh,
             idxs_v, idxd_v, i3s_a, i3d_a, i3s_b, i3d_b,
             rows0, rows1, rows2, rows3,
             ones_s_v, ones_d_v, zdeg_v, nsrc_v, ndst_v,
             sem_i, sem_g, sem_s,
             gsem0, gsem1, gsem2, gsem3, ssem0, ssem1, ssem2, ssem3):
    cid = lax.axis_index("c")
    sid = lax.axis_index("s")
    zf16 = jnp.zeros((16,), jnp.float32)
    zi16 = jnp.zeros((16,), jnp.int32)

    def _zero_row(r, _):
        for j in range(F // L):
            rows0[r, pl.ds(j * L, L)] = zf16
        return 0
    lax.fori_loop(0, NCH, _zero_row, 0)

    def _init_narrow(g, _):
        p = _iota16() + g * L
        r = lax.shift_right_logical(p, 3)
        c = jnp.bitwise_and(p, 7)
        one_s = jnp.where(c == 0, 1.0, 0.0).astype(jnp.float32)
        one_d = jnp.where(c == 1, 1.0, 0.0).astype(jnp.float32)
        plsc.store_scatter(ones_s_v, [r, c], one_s)
        plsc.store_scatter(ones_d_v, [r, c], one_d)
        return 0
    lax.fori_loop(0, CW * DW // L, _init_narrow, 0)

    def _zero_narrow(g, _):
        p = _iota16() + g * L
        r = lax.shift_right_logical(p, 3)
        c = jnp.bitwise_and(p, 7)
        plsc.store_scatter(zdeg_v, [r, c], zf16)
        return 0
    lax.fori_loop(0, NCH * DW // L, _zero_narrow, 0)

    def _zero_spmem(k, _):
        base = (sid + NS * k) * NCH
        pltpu.sync_copy(rows0, agg_sh.at[pl.ds(base, NCH)])
        pltpu.sync_copy(zdeg_v, deg_sh.at[pl.ds(base, NCH)])
        return 0
    lax.fori_loop(0, NODE_GP, _zero_spmem, 0)
    plsc.subcore_barrier()

    def _deg_group(k, _):
        row0 = (sid + NS * k) * KJ
        di1 = pltpu.async_copy(src_hbm.at[pl.ds(row0, KJ)], idxs_v, sem_i)
        di2 = pltpu.async_copy(dst_hbm.at[pl.ds(row0, KJ)], idxd_v, sem_i)
        di1.wait()
        di2.wait()
        descs = []
        for j in range(KJ):
            descs.append(pltpu.async_copy(
                ones_s_v, deg_sh.at[idxs_v.at[j]], sem_s, add=True))
            descs.append(pltpu.async_copy(
                ones_d_v, deg_sh.at[idxd_v.at[j]], sem_s, add=True))
        for d in descs:
            d.wait()
        return 0
    lax.fori_loop(0, DEG_GP, _deg_group, 0)
    plsc.subcore_barrier()

    def _node_chunk(k, _):
        base = (sid + NS * k) * NCH
        pltpu.sync_copy(deg_sh.at[pl.ds(base, NCH)], zdeg_v)
        for v in range(NCH // L):
            ridx = _iota16() + v * L
            do = plsc.load_gather(zdeg_v, [ridx, zi16])
            nsrc_v[pl.ds(k * NCH + v * L, L)] = _rsqrt16(jnp.maximum(do, 1.0))
            di = plsc.load_gather(zdeg_v, [ridx, zi16 + 1])
            ndst_v[pl.ds(k * NCH + v * L, L)] = _rsqrt16(jnp.maximum(di, 1.0))
        pltpu.sync_copy(x_hbm.at[pl.ds(base, NCH)], rows0)

        def _scale_grp(g, _):
            nv = nsrc_v[pl.ds(k * NCH + g * L, L)]
            for lane in range(L):
                s = nv[lane]
                r = g * L + lane
                for j in range(F // L):
                    rows0[r, pl.ds(j * L, L)] = rows0[r, pl.ds(j * L, L)] * s
            return 0
        lax.fori_loop(0, NCH // L, _scale_grp, 0)
        pltpu.sync_copy(rows0, feat_hbm.at[pl.ds(base, NCH)])
        return 0
    lax.fori_loop(0, NODE_GP, _node_chunk, 0)
    plsc.subcore_barrier()

    erow0 = cid * (EROWS3 // NC)
    ipair = ((i3s_a, i3d_a), (i3s_b, i3d_b))
    rbuf = (rows0, rows1, rows2, rows3)

    def _msg_idx(m):
        row0 = erow0 + (sid + NS * m) * KJ
        pa = ipair[m % 2]
        return [pltpu.async_copy(src3_hbm.at[pl.ds(row0, KJ)], pa[0], sem_i),
                pltpu.async_copy(dst3_hbm.at[pl.ds(row0, KJ)], pa[1], sem_i)]

    S = MSG_GP * KJ
    gsem = (gsem0, gsem1, gsem2, gsem3)
    ssem = (ssem0, ssem1, ssem2, ssem3)
    gd = {}
    sd = {}
    msg_id = {0: _msg_idx(0)}
    for d in msg_id[0]:
        d.wait()
    msg_id[1] = _msg_idx(1)
    for s0 in range(3):
        gd[s0] = pltpu.async_copy(
            feat_hbm.at[ipair[0][0].at[s0]], rbuf[s0], gsem[s0])
    for s in range(S):
        m, j = divmod(s, KJ)
        gd[s].wait()
        sd[s] = pltpu.async_copy(rbuf[s % 4],
                                 agg_sh.at[ipair[m % 2][1].at[j]],
                                 ssem[s % 4], add=True)
        if s + 3 < S:
            m3, j3 = divmod(s + 3, KJ)
            if s >= 1:
                sd[s - 1].wait()
            if j3 == 0:
                for d in msg_id[m3]:
                    d.wait()
            gd[s + 3] = pltpu.async_copy(
                feat_hbm.at[ipair[m3 % 2][0].at[j3]],
                rbuf[(s + 3) % 4], gsem[(s + 3) % 4])
        if j == 1 and 2 <= m + 1 < MSG_GP:
            msg_id[m + 1] = _msg_idx(m + 1)
    for st in range(S - 4, S):
        sd[st].wait()
    plsc.subcore_barrier()

    def _out_chunk(k, _):
        base = (sid + NS * k) * NCH
        pltpu.sync_copy(agg_sh.at[pl.ds(base, NCH)], rows0)

        def _scale_grp(g, _):
            nv = ndst_v[pl.ds(k * NCH + g * L, L)]
            for lane in range(L):
                s = nv[lane]
                r = g * L + lane
                for j in range(F // L):
                    rows0[r, pl.ds(j * L, L)] = rows0[r, pl.ds(j * L, L)] * s
            return 0
        lax.fori_loop(0, NCH // L, _scale_grp, 0)
        pltpu.sync_copy(rows0, part_hbm.at[pl.ds(cid * NP + base, NCH)])
        return 0
    lax.fori_loop(0, NODE_GP, _out_chunk, 0)


_sc_call = pl.kernel(
    _sc_body,
    out_type=(
        jax.ShapeDtypeStruct((NC * NP, F), jnp.float32),
        jax.ShapeDtypeStruct((NP, F), jnp.float32),
    ),
    mesh=plsc.VectorSubcoreMesh(core_axis_name="c", subcore_axis_name="s"),
    compiler_params=pltpu.CompilerParams(
        use_tc_tiling_on_sc=False, needs_layout_passes=False),
    scratch_types=[
        pltpu.VMEM_SHARED((NP, F), jnp.float32),
        pltpu.VMEM_SHARED((NP, DW), jnp.float32),
        pltpu.VMEM((KJ, CW), jnp.int32),
        pltpu.VMEM((KJ, CW), jnp.int32),
        pltpu.VMEM((KJ, CW3), jnp.int32),
        pltpu.VMEM((KJ, CW3), jnp.int32),
        pltpu.VMEM((KJ, CW3), jnp.int32),
        pltpu.VMEM((KJ, CW3), jnp.int32),
        pltpu.VMEM((CW3, F), jnp.float32),
        pltpu.VMEM((CW3, F), jnp.float32),
        pltpu.VMEM((CW3, F), jnp.float32),
        pltpu.VMEM((CW3, F), jnp.float32),
        pltpu.VMEM((CW, DW), jnp.float32),
        pltpu.VMEM((CW, DW), jnp.float32),
        pltpu.VMEM((NCH, DW), jnp.float32),
        pltpu.VMEM((NODE_GP * NCH,), jnp.float32),
        pltpu.VMEM((NODE_GP * NCH,), jnp.float32),
        pltpu.SemaphoreType.DMA,
        pltpu.SemaphoreType.DMA,
        pltpu.SemaphoreType.DMA,
        pltpu.SemaphoreType.DMA,
        pltpu.SemaphoreType.DMA,
        pltpu.SemaphoreType.DMA,
        pltpu.SemaphoreType.DMA,
        pltpu.SemaphoreType.DMA,
        pltpu.SemaphoreType.DMA,
        pltpu.SemaphoreType.DMA,
        pltpu.SemaphoreType.DMA,
    ],
)


def _tc_body(p0_ref, p1_ref, w_ref, b_ref, o_ref):
    acc = p0_ref[...] + p1_ref[...]
    y = jnp.dot(acc, w_ref[...], preferred_element_type=jnp.float32,
                precision=lax.Precision.HIGHEST)
    o_ref[...] = jax.nn.sigmoid(y + b_ref[...])


@jax.jit
def kernel(x, edge_index, W, b):
    pad = (jnp.arange(EP - E, dtype=jnp.int32) % (NP - N)) + N
    srcp = jnp.concatenate([edge_index[0], pad])
    dstp = jnp.concatenate([edge_index[1], pad])
    src3, dst3 = lax.optimization_barrier(
        (srcp.reshape(EROWS3, CW3), dstp.reshape(EROWS3, CW3)))
    xp = jnp.concatenate([x, jnp.zeros((NP - N, F), jnp.float32)], axis=0)
    part, _ = _sc_call(xp, srcp.reshape(EROWS, CW), dstp.reshape(EROWS, CW),
                       src3, dst3)
    out = pl.pallas_call(
        _tc_body,
        grid=(10,),
        in_specs=[
            pl.BlockSpec((N // 10, F), lambda i: (i, 0)),
            pl.BlockSpec((N // 10, F), lambda i: (i, 0)),
            pl.BlockSpec((F, F), lambda i: (0, 0)),
            pl.BlockSpec((1, F), lambda i: (0, 0)),
        ],
        out_specs=pl.BlockSpec((N // 10, F), lambda i: (i, 0)),
        out_shape=jax.ShapeDtypeStruct((N, F), jnp.float32),
    )(part[:N], part[NP:NP + N], W, b.reshape(1, F))
    return out
